# Initial kernel scaffold; baseline (speedup 1.0000x reference)
import jax
import jax.numpy as jnp
from jax.experimental import pallas as pl


def kernel(x, batch, u, W1, b1, W2, b2):
    raise NotImplementedError("write your pallas kernel here")


# speedup vs baseline: 3.3255x; 3.3255x over previous
import functools

import jax
import jax.numpy as jnp
from jax import lax
from jax.experimental import pallas as pl
from jax.experimental.pallas import tpu as pltpu
from jax.experimental.pallas import tpu_sc as plsc

NC = 2
NS = 16
NW = NC * NS
LANES = 16
CH = 128


def _tree_max(vals):
    while len(vals) > 1:
        nxt = [jnp.maximum(vals[i], vals[i + 1]) for i in range(0, len(vals) - 1, 2)]
        if len(vals) % 2:
            nxt.append(vals[-1])
        vals = nxt
    return vals[0]


def _make_segmax(n_rows, num_segments, d):
    ncg = d // LANES
    n_full = n_rows // CH
    tail_rows = n_rows - n_full * CH
    mesh = plsc.VectorSubcoreMesh(core_axis_name="c", subcore_axis_name="s",
                                  num_cores=NC, num_subcores=NS)

    def body(x_hbm, b_hbm, out_hbm, acc, xbuf, bbuf):
        wid = lax.axis_index("s") * NC + lax.axis_index("c")
        iota16 = lax.iota(jnp.int32, LANES)

        neg = jnp.full((LANES,), -jnp.inf, jnp.float32)
        def init_body(i, carry):
            for c in range(ncg):
                acc[i, pl.ds(c * LANES, LANES)] = neg
            return carry
        lax.fori_loop(0, num_segments, init_body, 0)

        def do_rowgroup(g):
            base = g * LANES
            bvec = bbuf[pl.ds(base, LANES)]
            bf = bvec.astype(jnp.float32)
            bmax = jnp.max(bf)
            bmin = jnp.min(bf)

            @pl.when(bmin == bmax)
            def _fast():
                svec = jnp.full((LANES,), bmin.astype(jnp.int32), jnp.int32)
                for c in range(ncg):
                    colidx = c * LANES + iota16
                    rows = [xbuf[base + r, pl.ds(c * LANES, LANES)]
                            for r in range(LANES)]
                    m = _tree_max(rows)
                    old = plsc.load_gather(acc, [svec, colidx])
                    plsc.store_scatter(acc, [svec, colidx], jnp.maximum(old, m))

            @pl.when(bmin != bmax)
            def _mixed():
                for j in range(LANES):
                    sj = jnp.max(jnp.where(iota16 == j, bf, -1.0))
                    svec = jnp.full((LANES,), sj.astype(jnp.int32), jnp.int32)
                    for c in range(ncg):
                        colidx = c * LANES + iota16
                        v = xbuf[base + j, pl.ds(c * LANES, LANES)]
                        old = plsc.load_gather(acc, [svec, colidx])
                        plsc.store_scatter(acc, [svec, colidx],
                                           jnp.maximum(old, v))

        start_c = (wid * n_full) // NW
        end_c = ((wid + 1) * n_full) // NW

        def chunk_body(c, carry):
            pltpu.sync_copy(x_hbm.at[pl.ds(c * CH, CH)], xbuf)
            pltpu.sync_copy(b_hbm.at[pl.ds(c * CH, CH)], bbuf)
            def g_body(g, carry2):
                do_rowgroup(g)
                return carry2
            lax.fori_loop(0, CH // LANES, g_body, 0)
            return carry
        lax.fori_loop(start_c, end_c, chunk_body, 0)

        if tail_rows:
            @pl.when(wid == NW - 1)
            def _tail():
                t0 = n_full * CH
                pltpu.sync_copy(x_hbm.at[pl.ds(t0, tail_rows)],
                                xbuf.at[pl.ds(0, tail_rows)])
                pltpu.sync_copy(b_hbm.at[pl.ds(t0, tail_rows)],
                                bbuf.at[pl.ds(0, tail_rows)])
                def tg_body(g, carry2):
                    do_rowgroup(g)
                    return carry2
                lax.fori_loop(0, tail_rows // LANES, tg_body, 0)

        pltpu.sync_copy(acc, out_hbm.at[wid])

    return pl.kernel(
        body,
        out_type=jax.ShapeDtypeStruct((NW, num_segments, d), jnp.float32),
        mesh=mesh,
        compiler_params=pltpu.CompilerParams(needs_layout_passes=False),
        scratch_types=[
            pltpu.VMEM((num_segments, d), jnp.float32),
            pltpu.VMEM((CH, d), jnp.float32),
            pltpu.VMEM((CH,), jnp.int32),
        ],
    )


def _mlp_body(parts_ref, u_ref, w1a_ref, w1b_ref, b1_ref, w2_ref, b2_ref,
              out_ref):
    agg = parts_ref[0]
    for i in range(1, NW):
        agg = jnp.maximum(agg, parts_ref[i])
    agg = jnp.where(jnp.isfinite(agg), agg, 0.0)
    h = (jnp.dot(u_ref[...], w1a_ref[...], preferred_element_type=jnp.float32)
         + jnp.dot(agg, w1b_ref[...], preferred_element_type=jnp.float32)
         + b1_ref[...])
    h = jnp.maximum(h, 0.0)
    out_ref[...] = (jnp.dot(h, w2_ref[...], preferred_element_type=jnp.float32)
                    + b2_ref[...])


@jax.jit
def kernel(x, batch, u, W1, b1, W2, b2):
    n, d = x.shape
    num_segments = u.shape[0]
    batch = batch.astype(jnp.int32)
    parts = _make_segmax(n, num_segments, d)(x, batch)
    mlp = pl.pallas_call(
        _mlp_body,
        out_shape=jax.ShapeDtypeStruct((num_segments, d), jnp.float32),
    )
    return mlp(parts, u, W1[:d], W1[d:], b1.reshape(1, d), W2,
               b2.reshape(1, d))


# speedup vs baseline: 5.3987x; 1.6234x over previous
import functools

import jax
import jax.numpy as jnp
from jax import lax
from jax.experimental import pallas as pl
from jax.experimental.pallas import tpu as pltpu
from jax.experimental.pallas import tpu_sc as plsc

NC = 2
NS = 16
NW = NC * NS
LANES = 16
CH = 160


def _tree_max(vals):
    while len(vals) > 1:
        nxt = [jnp.maximum(vals[i], vals[i + 1]) for i in range(0, len(vals) - 1, 2)]
        if len(vals) % 2:
            nxt.append(vals[-1])
        vals = nxt
    return vals[0]


def _make_segmax(n_rows, num_segments, d):
    ncg = d // LANES
    n_chunks = n_rows // CH
    assert n_chunks * CH == n_rows
    mesh = plsc.VectorSubcoreMesh(core_axis_name="c", subcore_axis_name="s",
                                  num_cores=NC, num_subcores=NS)

    def body(x_hbm, b_hbm, out_hbm, acc, xbuf, bbuf, semx, semb):
        wid = lax.axis_index("s") * NC + lax.axis_index("c")
        iota16 = lax.iota(jnp.int32, LANES)

        neg = jnp.full((LANES,), -jnp.inf, jnp.float32)
        def init_body(i, carry):
            for c in range(ncg):
                acc[i, pl.ds(c * LANES, LANES)] = neg
            return carry

        def do_rowgroup(par, g):
            base = g * LANES
            bvec = bbuf[pl.ds(par * CH + base, LANES)]
            bf = bvec.astype(jnp.float32)
            bmax = jnp.max(bf)
            bmin = jnp.min(bf)

            @pl.when(bmin == bmax)
            def _fast():
                svec = jnp.full((LANES,), bmin.astype(jnp.int32), jnp.int32)
                for c in range(ncg):
                    colidx = c * LANES + iota16
                    rows = [xbuf[par, base + r, pl.ds(c * LANES, LANES)]
                            for r in range(LANES)]
                    m = _tree_max(rows)
                    old = plsc.load_gather(acc, [svec, colidx])
                    plsc.store_scatter(acc, [svec, colidx], jnp.maximum(old, m))

            @pl.when(bmin != bmax)
            def _mixed():
                for j in range(LANES):
                    sj = jnp.max(jnp.where(iota16 == j, bf, -1.0))
                    svec = jnp.full((LANES,), sj.astype(jnp.int32), jnp.int32)
                    for c in range(ncg):
                        colidx = c * LANES + iota16
                        v = xbuf[par, base + j, pl.ds(c * LANES, LANES)]
                        old = plsc.load_gather(acc, [svec, colidx])
                        plsc.store_scatter(acc, [svec, colidx],
                                           jnp.maximum(old, v))

        start_c = (wid * n_chunks) // NW
        end_c = ((wid + 1) * n_chunks) // NW
        n_my = end_c - start_c

        def start_fetch(i, slot):
            c = start_c + i
            pltpu.async_copy(x_hbm.at[pl.ds(c * CH, CH)], xbuf.at[slot],
                             semx.at[slot])
            pltpu.async_copy(b_hbm.at[pl.ds(c * CH, CH)],
                             bbuf.at[pl.ds(slot * CH, CH)], semb.at[slot])

        @pl.when(n_my > 0)
        def _prime():
            start_fetch(0, 0)

        lax.fori_loop(0, num_segments, init_body, 0)

        def chunk_body(i, carry):
            cur = lax.rem(i, 2)
            pltpu.make_async_copy(x_hbm.at[pl.ds(0, CH)], xbuf.at[cur],
                                  semx.at[cur]).wait()
            pltpu.make_async_copy(b_hbm.at[pl.ds(0, CH)],
                                  bbuf.at[pl.ds(cur * CH, CH)],
                                  semb.at[cur]).wait()

            @pl.when(i + 1 < n_my)
            def _prefetch():
                start_fetch(i + 1, 1 - cur)

            def g_body(g, carry2):
                do_rowgroup(cur, g)
                return carry2
            lax.fori_loop(0, CH // LANES, g_body, 0)
            return carry
        lax.fori_loop(0, n_my, chunk_body, 0)

        pltpu.sync_copy(acc, out_hbm.at[wid])

    return pl.kernel(
        body,
        out_type=jax.ShapeDtypeStruct((NW, num_segments, d), jnp.float32),
        mesh=mesh,
        compiler_params=pltpu.CompilerParams(needs_layout_passes=False),
        scratch_types=[
            pltpu.VMEM((num_segments, d), jnp.float32),
            pltpu.VMEM((2, CH, d), jnp.float32),
            pltpu.VMEM((2 * CH,), jnp.int32),
            pltpu.SemaphoreType.DMA((2,)),
            pltpu.SemaphoreType.DMA((2,)),
        ],
    )


def _mlp_body(parts_ref, u_ref, w1a_ref, w1b_ref, b1_ref, w2_ref, b2_ref,
              out_ref):
    agg = parts_ref[0]
    for i in range(1, NW):
        agg = jnp.maximum(agg, parts_ref[i])
    agg = jnp.where(jnp.isfinite(agg), agg, 0.0)
    h = (jnp.dot(u_ref[...], w1a_ref[...], preferred_element_type=jnp.float32)
         + jnp.dot(agg, w1b_ref[...], preferred_element_type=jnp.float32)
         + b1_ref[...])
    h = jnp.maximum(h, 0.0)
    out_ref[...] = (jnp.dot(h, w2_ref[...], preferred_element_type=jnp.float32)
                    + b2_ref[...])


@jax.jit
def kernel(x, batch, u, W1, b1, W2, b2):
    n, d = x.shape
    num_segments = u.shape[0]
    batch = batch.astype(jnp.int32)
    parts = _make_segmax(n, num_segments, d)(x, batch)
    mlp = pl.pallas_call(
        _mlp_body,
        out_shape=jax.ShapeDtypeStruct((num_segments, d), jnp.float32),
    )
    return mlp(parts, u, W1[:d], W1[d:], b1.reshape(1, d), W2,
               b2.reshape(1, d))


# speedup vs baseline: 5.6982x; 1.0555x over previous
import functools

import jax
import jax.numpy as jnp
from jax import lax
from jax.experimental import pallas as pl
from jax.experimental.pallas import tpu as pltpu
from jax.experimental.pallas import tpu_sc as plsc

NC = 2
NS = 16
NW = NC * NS
LANES = 16
CH = 160


def _tree_max(vals):
    while len(vals) > 1:
        nxt = [jnp.maximum(vals[i], vals[i + 1]) for i in range(0, len(vals) - 1, 2)]
        if len(vals) % 2:
            nxt.append(vals[-1])
        vals = nxt
    return vals[0]


def _make_segmax(n_rows, num_segments, d):
    ncg = d // LANES
    n_chunks = n_rows // CH
    assert n_chunks * CH == n_rows
    max_my = -(-n_chunks // NW)
    pre_rows = max_my * CH
    mesh = plsc.VectorSubcoreMesh(core_axis_name="c", subcore_axis_name="s",
                                  num_cores=NC, num_subcores=NS)

    def body(x_hbm, b_hbm, out_hbm, acc, xbuf, bbuf, semx, semb):
        wid = lax.axis_index("s") * NC + lax.axis_index("c")

        start_c = (wid * n_chunks) // NW
        end_c = ((wid + 1) * n_chunks) // NW
        n_my = end_c - start_c

        def start_fetch(i, slot):
            c = start_c + i
            pltpu.async_copy(x_hbm.at[pl.ds(c * CH, CH)], xbuf.at[slot],
                             semx.at[slot])

        pltpu.async_copy(b_hbm.at[pl.ds(start_c * CH, pre_rows)], bbuf, semb)

        @pl.when(n_my > 0)
        def _prime():
            start_fetch(0, 0)

        neg = jnp.full((LANES,), -jnp.inf, jnp.float32)
        def init_body(i, carry):
            for c in range(ncg):
                acc[i, pl.ds(c * LANES, LANES)] = neg
            return carry
        lax.fori_loop(0, num_segments, init_body, 0)

        pltpu.make_async_copy(b_hbm.at[pl.ds(0, pre_rows)], bbuf, semb).wait()

        def do_rowgroup(par, i, g):
            base = g * LANES
            bvec = bbuf[pl.ds(i * CH + base, LANES)]
            s0 = bvec[0]
            s15 = bvec[LANES - 1]

            @pl.when(s0 == s15)
            def _fast():
                for c in range(ncg):
                    cs = pl.ds(c * LANES, LANES)
                    rows = [xbuf[par, base + r, cs] for r in range(LANES)]
                    m = _tree_max(rows)
                    acc[s0, cs] = jnp.maximum(acc[s0, cs], m)

            @pl.when(s0 != s15)
            def _mixed():
                for j in range(LANES):
                    sj = bvec[j]
                    for c in range(ncg):
                        cs = pl.ds(c * LANES, LANES)
                        v = xbuf[par, base + j, cs]
                        acc[sj, cs] = jnp.maximum(acc[sj, cs], v)

        def chunk_body(i, carry):
            cur = lax.rem(i, 2)
            pltpu.make_async_copy(x_hbm.at[pl.ds(0, CH)], xbuf.at[cur],
                                  semx.at[cur]).wait()

            @pl.when(i + 1 < n_my)
            def _prefetch():
                start_fetch(i + 1, 1 - cur)

            def g_body(g, carry2):
                do_rowgroup(cur, i, 2 * g)
                do_rowgroup(cur, i, 2 * g + 1)
                return carry2
            lax.fori_loop(0, CH // LANES // 2, g_body, 0)
            return carry
        lax.fori_loop(0, n_my, chunk_body, 0)

        pltpu.sync_copy(acc, out_hbm.at[wid])

    return pl.kernel(
        body,
        out_type=jax.ShapeDtypeStruct((NW, num_segments, d), jnp.float32),
        mesh=mesh,
        compiler_params=pltpu.CompilerParams(needs_layout_passes=False),
        scratch_types=[
            pltpu.VMEM((num_segments, d), jnp.float32),
            pltpu.VMEM((2, CH, d), jnp.float32),
            pltpu.VMEM((pre_rows,), jnp.int32),
            pltpu.SemaphoreType.DMA((2,)),
            pltpu.SemaphoreType.DMA,
        ],
    )


def _mlp_body(parts_ref, u_ref, w1a_ref, w1b_ref, b1_ref, w2_ref, b2_ref,
              out_ref):
    agg = parts_ref[0]
    for i in range(1, NW):
        agg = jnp.maximum(agg, parts_ref[i])
    agg = jnp.where(jnp.isfinite(agg), agg, 0.0)
    h = (jnp.dot(u_ref[...], w1a_ref[...], preferred_element_type=jnp.float32)
         + jnp.dot(agg, w1b_ref[...], preferred_element_type=jnp.float32)
         + b1_ref[...])
    h = jnp.maximum(h, 0.0)
    out_ref[...] = (jnp.dot(h, w2_ref[...], preferred_element_type=jnp.float32)
                    + b2_ref[...])


@jax.jit
def kernel(x, batch, u, W1, b1, W2, b2):
    n, d = x.shape
    num_segments = u.shape[0]
    batch = batch.astype(jnp.int32)
    parts = _make_segmax(n, num_segments, d)(x, batch)
    mlp = pl.pallas_call(
        _mlp_body,
        out_shape=jax.ShapeDtypeStruct((num_segments, d), jnp.float32),
    )
    return mlp(parts, u, W1[:d], W1[d:], b1.reshape(1, d), W2,
               b2.reshape(1, d))
